# baseline (speedup 1.0000x reference)
import jax
import jax.numpy as jnp
import numpy as np
from jax.experimental import pallas as pl

B = 16
H = 224
W = 224
PATCH = 14
PH = H // PATCH
PW = W // PATCH
P = PH * PW
D = 1024
L = 2
NF = 4
NR = B - 1
NS = B // 2
KSEL = 5


def _pool_matrix() -> np.ndarray:
    a = np.zeros((PH, PH), np.float32)
    for i in range(PH):
        lo, hi = max(0, i - 1), min(PH - 1, i + 1)
        a[i, lo:hi + 1] = 1.0 / (hi - lo + 1)
    return np.kron(a, a).astype(np.float32)


def _resize_matrix() -> np.ndarray:
    scale = H / PH
    out = np.zeros((H, PH), np.float32)
    for i in range(H):
        x = (i + 0.5) / scale - 0.5
        w = np.maximum(0.0, 1.0 - np.abs(x - np.arange(PH)))
        out[i] = w / w.sum()
    return out


_POOL = _pool_matrix()
_RESIZE = _resize_matrix()


def _k1_body(patches_ref, wp_ref, bp_ref, wl_ref, bl_ref, pool_ref, feats_ref):
    t = jax.lax.dot_general(
        patches_ref[0], wp_ref[...], (((1,), (0,)), ((), ())),
        preferred_element_type=jnp.float32) + bp_ref[...]
    pool = pool_ref[...]
    for l in range(L):
        x = jax.lax.dot_general(
            t, wl_ref[l], (((1,), (0,)), ((), ())),
            preferred_element_type=jnp.float32) + bl_ref[l]
        f = jax.nn.gelu(x)
        feats_ref[l, 0] = f
        feats_ref[2 + l, 0] = jax.lax.dot_general(
            pool, f, (((1,), (0,)), ((), ())),
            preferred_element_type=jnp.float32)


def _k2_body(q_ref, r_ref, ma_ref, mb_ref):
    q = q_ref[0, 0]
    r = r_ref[0, 0]
    s = jax.lax.dot_general(q, r, (((1,), (1,)), ((), ())),
                            preferred_element_type=jnp.float32)
    sqq = jnp.sum(q * q, axis=1)
    sqr = jnp.sum(r * r, axis=1)
    rowmin = sqq + jnp.min(sqr[None, :] - 2.0 * s, axis=1)
    colmin = sqr + jnp.min(sqq[:, None] - 2.0 * s, axis=0)
    ma_ref[0, 0, 0] = jnp.sqrt(jnp.maximum(rowmin, 1e-12))
    mb_ref[0, 0, 0] = jnp.sqrt(jnp.maximum(colmin, 1e-12))


def _k3a_body(ma_ref, mb_ref, scores_ref):
    ma = ma_ref[...]
    mb = mb_ref[...]
    img = jax.lax.broadcasted_iota(jnp.int32, ma.shape, 3) // P
    rnd = jax.lax.broadcasted_iota(jnp.int32, ma.shape, 1)
    side = (img - rnd) % NR
    is_a = (img == B - 1) | ((side >= 1) & (side <= NS - 1))
    vals = jnp.where(is_a, ma, mb)
    total = jnp.zeros((NF, 1, 1, B * P), jnp.float32)
    big = jnp.float32(3.0e38)
    for _ in range(KSEL):
        mv = jnp.min(vals, axis=1, keepdims=True)
        total = total + mv
        eq = vals <= mv
        idx = jnp.where(eq, rnd, NR)
        first = rnd == jnp.min(idx, axis=1, keepdims=True)
        vals = jnp.where(first, big, vals)
    scores_ref[...] = jnp.mean(total, axis=0)[0] * (1.0 / KSEL)


def _k3b_body(scores_ref, m_ref, pix_ref, final_ref):
    g = scores_ref[0]
    final_ref[0, 0] = jnp.full((128,), jnp.max(g), jnp.float32)
    m = m_ref[...]
    a1 = jax.lax.dot_general(m, g, (((1,), (0,)), ((), ())),
                             preferred_element_type=jnp.float32)
    pix_ref[0] = jax.lax.dot_general(a1, m, (((1,), (1,)), ((), ())),
                                     preferred_element_type=jnp.float32)


@jax.jit
def kernel(pixel_values, W_patch, b_patch, W_layers, b_layers):
    patches = pixel_values.reshape(B, 3, PH, PATCH, PW, PATCH)
    patches = patches.transpose(0, 2, 4, 1, 3, 5).reshape(B, P, 3 * PATCH * PATCH)
    cdim = patches.shape[-1]

    feats = pl.pallas_call(
        _k1_body,
        grid=(B,),
        in_specs=[
            pl.BlockSpec((1, P, cdim), lambda b: (b, 0, 0)),
            pl.BlockSpec((cdim, D), lambda b: (0, 0)),
            pl.BlockSpec((D,), lambda b: (0,)),
            pl.BlockSpec((L, D, D), lambda b: (0, 0, 0)),
            pl.BlockSpec((L, D), lambda b: (0, 0)),
            pl.BlockSpec((P, P), lambda b: (0, 0)),
        ],
        out_specs=pl.BlockSpec((NF, 1, P, D), lambda b: (0, b, 0, 0)),
        out_shape=jax.ShapeDtypeStruct((NF, B, P, D), jnp.float32),
    )(patches, W_patch, b_patch, W_layers, b_layers, jnp.asarray(_POOL))

    def _a_idx(r, s):
        return jnp.where(s == 0, B - 1, (r + s) % NR)

    def _b_idx(r, s):
        return (r - s) % NR

    ma, mb = pl.pallas_call(
        _k2_body,
        grid=(NF, NR, NS),
        in_specs=[
            pl.BlockSpec((1, 1, P, D), lambda f, r, s: (f, _a_idx(r, s), 0, 0)),
            pl.BlockSpec((1, 1, P, D), lambda f, r, s: (f, _b_idx(r, s), 0, 0)),
        ],
        out_specs=[
            pl.BlockSpec((1, 1, 1, P), lambda f, r, s: (f, r, 0, _a_idx(r, s))),
            pl.BlockSpec((1, 1, 1, P), lambda f, r, s: (f, r, 0, _b_idx(r, s))),
        ],
        out_shape=[
            jax.ShapeDtypeStruct((NF, NR, 1, B * P), jnp.float32),
            jax.ShapeDtypeStruct((NF, NR, 1, B * P), jnp.float32),
        ],
    )(feats, feats)

    scores = pl.pallas_call(
        _k3a_body,
        in_specs=[
            pl.BlockSpec((NF, NR, 1, B * P), lambda: (0, 0, 0, 0)),
            pl.BlockSpec((NF, NR, 1, B * P), lambda: (0, 0, 0, 0)),
        ],
        out_specs=pl.BlockSpec((1, B * P), lambda: (0, 0)),
        out_shape=jax.ShapeDtypeStruct((1, B * P), jnp.float32),
    )(ma, mb)

    pix, final = pl.pallas_call(
        _k3b_body,
        grid=(B,),
        in_specs=[
            pl.BlockSpec((1, PH, PW), lambda b: (b, 0, 0)),
            pl.BlockSpec((H, PH), lambda b: (0, 0)),
        ],
        out_specs=[
            pl.BlockSpec((1, H, W), lambda b: (b, 0, 0)),
            pl.BlockSpec((1, 1, 128), lambda b: (b, 0, 0)),
        ],
        out_shape=[
            jax.ShapeDtypeStruct((B, H, W), jnp.float32),
            jax.ShapeDtypeStruct((B, 1, 128), jnp.float32),
        ],
    )(scores.reshape(B, PH, PW), jnp.asarray(_RESIZE))

    return final[:, 0, 0], pix
